# baseline (speedup 1.0000x reference)
import jax
import jax.numpy as jnp
from jax.experimental import pallas as pl
from jax.experimental.pallas import tpu as pltpu

_ROWS = 16384
_K = 2048
_KSTRIP = 256
_NSTRIP = _K // _KSTRIP
_NBUF = 2


def _router_kernel(x_hbm, w_ref, o_ref, x_vmem, sems):
    def make_copy(k, slot):
        return pltpu.make_async_copy(
            x_hbm.at[:, pl.ds(k * _KSTRIP, _KSTRIP)],
            x_vmem.at[slot],
            sems.at[slot],
        )

    for k in range(_NBUF):
        make_copy(k, k).start()

    for k in range(_NSTRIP):
        slot = k % _NBUF
        make_copy(k, slot).wait()
        wk = w_ref[:, k * _KSTRIP:(k + 1) * _KSTRIP].astype(jnp.bfloat16)
        part = jax.lax.dot_general(
            x_vmem[slot].astype(jnp.bfloat16), wk,
            dimension_numbers=(((1,), (1,)), ((), ())),
            preferred_element_type=jnp.float32,
        )
        if k == 0:
            o_ref[...] = part
        elif k < _NSTRIP - 1:
            o_ref[...] += part
        else:
            o_ref[...] = jnp.maximum(o_ref[...] + part, 0.0)
        if k + _NBUF < _NSTRIP:
            make_copy(k + _NBUF, slot).start()


def kernel(x, W):
    M, K = x.shape
    E = W.shape[0]
    return pl.pallas_call(
        _router_kernel,
        in_specs=[
            pl.BlockSpec(memory_space=pl.ANY),
            pl.BlockSpec((E, K), lambda: (0, 0)),
        ],
        out_specs=pl.BlockSpec((M, E), lambda: (0, 0)),
        out_shape=jax.ShapeDtypeStruct((M, E), x.dtype),
        scratch_shapes=[
            pltpu.VMEM((_NBUF, _ROWS, _KSTRIP), jnp.float32),
            pltpu.SemaphoreType.DMA((_NBUF,)),
        ],
    )(x, W)


# speedup vs baseline: 1.1230x; 1.1230x over previous
import jax
import jax.numpy as jnp
from jax.experimental import pallas as pl


def _router_kernel(x_ref, w_ref, o_ref):
    logits = jax.lax.dot_general(
        x_ref[...].astype(jnp.bfloat16), w_ref[...].astype(jnp.bfloat16),
        dimension_numbers=(((1,), (1,)), ((), ())),
        preferred_element_type=jnp.float32,
    )
    o_ref[...] = jnp.maximum(logits, 0.0)


def kernel(x, W):
    M, K = x.shape
    E = W.shape[0]
    BM = 1024
    return pl.pallas_call(
        _router_kernel,
        grid=(M // BM,),
        in_specs=[
            pl.BlockSpec((BM, K), lambda i: (i, 0)),
            pl.BlockSpec((E, K), lambda i: (0, 0)),
        ],
        out_specs=pl.BlockSpec((BM, E), lambda i: (i, 0)),
        out_shape=jax.ShapeDtypeStruct((M, E), x.dtype),
    )(x, W)
